# baseline (speedup 1.0000x reference)
import jax
import jax.numpy as jnp
from jax.experimental import pallas as pl
from jax.experimental.pallas import tpu as pltpu

_NUM_CHUNKS = 8


def _dma_copy(in_ref, out_ref, sems):
    rows = out_ref.shape[0]
    chunk = rows // _NUM_CHUNKS
    copies = [
        pltpu.make_async_copy(
            in_ref.at[pl.ds(i * chunk, chunk), :],
            out_ref.at[pl.ds(i * chunk, chunk), :],
            sems.at[i],
        )
        for i in range(_NUM_CHUNKS)
    ]
    for c in copies:
        c.start()
    for c in copies:
        c.wait()


def kernel(x, pos_emb):
    seq_len = x.shape[1]
    d_model = pos_emb.shape[1]
    return pl.pallas_call(
        _dma_copy,
        in_specs=[pl.BlockSpec(memory_space=pl.ANY)],
        out_specs=pl.BlockSpec(memory_space=pl.ANY),
        scratch_shapes=[pltpu.SemaphoreType.DMA((_NUM_CHUNKS,))],
        out_shape=jax.ShapeDtypeStruct((seq_len, d_model), pos_emb.dtype),
    )(pos_emb)


# speedup vs baseline: 49.0516x; 49.0516x over previous
import jax
import jax.numpy as jnp
from jax.experimental import pallas as pl
from jax.experimental.pallas import tpu as pltpu


def _copy_block(in_ref, out_ref):
    out_ref[...] = in_ref[...]


def kernel(x, pos_emb):
    seq_len = x.shape[1]
    d_model = pos_emb.shape[1]
    block_rows = 2048
    num_blocks = pl.cdiv(seq_len, block_rows)
    return pl.pallas_call(
        _copy_block,
        grid=(num_blocks,),
        in_specs=[pl.BlockSpec((block_rows, d_model), lambda i: (i, 0))],
        out_specs=pl.BlockSpec((block_rows, d_model), lambda i: (i, 0)),
        out_shape=jax.ShapeDtypeStruct((seq_len, d_model), pos_emb.dtype),
        compiler_params=pltpu.CompilerParams(
            dimension_semantics=("parallel",),
        ),
    )(pos_emb)
